# baseline (speedup 1.0000x reference)
import functools

import jax
import jax.numpy as jnp
from jax import lax
from jax.experimental import pallas as pl
from jax.experimental.pallas import tpu as pltpu
from jax.experimental.pallas import tpu_sc as plsc

CUTOFF = 5.0

NC = 2
NS = 16
BN = 1000
EB = 2000
K = 32
KD = 40


def _prep_mats(h, w, t_ref):
    dot = functools.partial(jnp.dot, preferred_element_type=jnp.float32,
                            precision=lax.Precision.HIGHEST)
    t_ref[0] = dot(h, w[:128, :128].T)
    t_ref[1] = dot(h, w[:128, 128:].T)
    t_ref[2] = dot(h, w[128:, :128].T)
    t_ref[3] = dot(h, w[128:, 128:].T)


def _embed_prep_body(nf_ref, emb_ref, w_ref, h_ref, t_ref):
    nf = nf_ref[...]
    v = emb_ref.shape[0]
    io = lax.broadcasted_iota(jnp.int32, (nf.shape[0], v), 1)
    onehot = jnp.where(nf == io, 1.0, 0.0).astype(jnp.float32)
    h = jnp.dot(onehot, emb_ref[...], preferred_element_type=jnp.float32,
                precision=lax.Precision.HIGHEST)
    h_ref[...] = h
    _prep_mats(h, w_ref[...], t_ref)


def _rmat_body(ea_ref, w1c_ref, b1_ref, out_ref, *, nrbf):
    ea = ea_ref[...]
    d = jnp.sqrt(jnp.sum(ea * ea, axis=1, keepdims=True))
    mu = lax.broadcasted_iota(jnp.int32, (1, nrbf), 1).astype(jnp.float32) * (
        CUTOFF / (nrbf - 1))
    sigma = CUTOFF / nrbf
    rbf = jnp.exp(-((d - mu) ** 2) * (1.0 / (2.0 * sigma * sigma)))
    w1c = w1c_ref[0]
    r = jnp.dot(rbf, w1c.T, preferred_element_type=jnp.float32,
                precision=lax.Precision.HIGHEST) + b1_ref[0]
    out_ref[0, 0] = r[:, :128]
    out_ref[0, 1] = r[:, 128:]


def _prep_body(h_ref, w_ref, out_ref):
    h = h_ref[...]
    w = w_ref[...]
    dot = functools.partial(jnp.dot, preferred_element_type=jnp.float32,
                            precision=lax.Precision.HIGHEST)
    out_ref[0] = dot(h, w[:128, :128].T)
    out_ref[1] = dot(h, w[:128, 128:].T)
    out_ref[2] = dot(h, w[128:, :128].T)
    out_ref[3] = dot(h, w[128:, 128:].T)


def _silu(x):
    return x * (1.0 / (1.0 + jnp.exp(-x)))


def _update_body(h_ref, agg_ref, degs_ref, w2_ref, b2_ref, uw1_ref, ub1_ref,
                 uw2_ref, ub2_ref, out_ref, *extra):
    h = h_ref[...]
    a0 = agg_ref[0]
    a1 = agg_ref[1]
    deg = degs_ref[0][:, 0:1] + degs_ref[1][:, 0:1]
    dot = functools.partial(jnp.dot, preferred_element_type=jnp.float32,
                            precision=lax.Precision.HIGHEST)
    w2 = w2_ref[...]
    agg = (dot(jnp.concatenate([a0, a1], axis=1), w2.T)
           + deg * b2_ref[...])
    uw1 = uw1_ref[...]
    pre = (dot(jnp.concatenate([h, agg], axis=1), uw1.T) + ub1_ref[...])
    hid = _silu(pre)
    hnew = h + dot(hid, uw2_ref[...].T) + ub2_ref[...]
    if extra:
        hout_ref, tout_ref = extra
        hout_ref[...] = hnew
        _prep_mats(hnew, out_ref[...], tout_ref)
    else:
        out_ref[...] = hnew


def _energy_body(h_ref, ow1_ref, ob1_ref, ow2_ref, ob2_ref, out_ref, *, n):
    dot = functools.partial(jnp.dot, preferred_element_type=jnp.float32,
                            precision=lax.Precision.HIGHEST)
    pre = dot(h_ref[...], ow1_ref[...].T) + ob1_ref[...]
    hid = _silu(pre)
    ae = dot(hid, ow2_ref[...].T)
    @pl.when(pl.program_id(0) == 0)
    def _():
        out_ref[...] = ob2_ref[...] * float(n)
    out_ref[...] += jnp.sum(ae).reshape(1, 1)


def _edge_body(t_ref, r_ref, idx2_ref, out_ref,
               ib, idst, gbuf, rbuf, agg, semg, semr, *, n, e, npad):
    cid = lax.axis_index("c")
    sid = lax.axis_index("s")
    ep = e // NS
    nch = ep // K
    rows_pt = npad // NS
    nzc = rows_pt // (2 * K)

    def _zrow(i, _):
        for v in range(8):
            gbuf[0][i, pl.ds(v * 16, 16)] = jnp.zeros((16,), jnp.float32)
        return 0
    lax.fori_loop(0, 2 * K, _zrow, 0)
    for q in range(nzc):
        pltpu.sync_copy(gbuf[0],
                        agg.at[pl.ds(sid * rows_pt + q * 2 * K, 2 * K)])
    plsc.subcore_barrier()

    off_a = cid * (2 * n)
    off_b = off_a + n
    base = sid * ep

    def _issue(j, b):
        g0 = pl.multiple_of((base // K + j) * 2 * K, 16)
        pltpu.sync_copy(idx2_ref.at[pl.ds(g0, 2 * K)], ib[b])
        for t in range(K // 16):
            sl = pl.ds(t * 16, 16)
            idst[b][sl] = ib[b][pl.ds(K + t * 16, 16)]
        for t in range(2 * K // 16):
            sl = pl.ds(t * 16, 16)
            ib[b][sl] = ib[b][sl] + (off_a if t < K // 16 else off_b)
        pltpu.async_copy(t_ref.at[ib[b]], gbuf[b], semg[b])
        e0 = pl.multiple_of(base + j * K, 16)
        pltpu.async_copy(r_ref.at[cid, pl.ds(e0, K)], rbuf[b], semr[b])

    def _process(b):
        pltpu.make_async_copy(t_ref.at[ib[b]], gbuf[b], semg[b]).wait()
        pltpu.make_async_copy(r_ref.at[cid, pl.ds(0, K)], rbuf[b],
                              semr[b]).wait()

        def _row(i, _):
            for v in range(8):
                sl = pl.ds(v * 16, 16)
                x = gbuf[b][i, sl] + gbuf[b][K + i, sl] + rbuf[b][i, sl]
                gbuf[b][i, sl] = x * (1.0 / (1.0 + jnp.exp(-x)))
            return 0
        lax.fori_loop(0, K, _row, 0)
        pltpu.sync_copy(gbuf[b].at[pl.ds(0, K)], agg.at[idst[b]], add=True)

    _issue(0, 0)
    _issue(1, 1)

    def _grp(jj, _):
        j = jj * 3
        for p in range(3):
            c = j + p

            @pl.when(c + 2 < nch)
            def _():
                _issue(c + 2, (p + 2) % 3)

            @pl.when(c < nch)
            def _():
                _process(p)
        return 0
    lax.fori_loop(0, (nch + 2) // 3, _grp, 0)
    plsc.subcore_barrier()

    for q in range(nzc):
        rows = pl.ds(sid * rows_pt + q * 2 * K, 2 * K)
        pltpu.sync_copy(agg.at[rows], gbuf[0])
        pltpu.sync_copy(gbuf[0], out_ref.at[cid, rows])


def _deg_body(dst_ref, out_ref, zb, ones, idx, degsp, *, n, e, npad):
    cid = lax.axis_index("c")
    sid = lax.axis_index("s")
    ept = e // (NC * NS)
    nch = ept // KD
    rows_pt = npad // NS
    zch = rows_pt // 5

    def _z(i, _):
        for v in range(8):
            zb[i, pl.ds(v * 16, 16)] = jnp.zeros((16,), jnp.float32)
        return 0
    lax.fori_loop(0, zch, _z, 0)

    def _o(i, _):
        for v in range(8):
            ones[i, pl.ds(v * 16, 16)] = jnp.full((16,), 1.0, jnp.float32)
        return 0
    lax.fori_loop(0, KD, _o, 0)

    for q in range(5):
        pltpu.sync_copy(zb, degsp.at[pl.ds(sid * rows_pt + q * zch, zch)])
    plsc.subcore_barrier()

    base = cid * (e // NC) + sid * ept

    def _chunk(j, _):
        e0 = pl.multiple_of(base + j * KD, 8)
        pltpu.sync_copy(dst_ref.at[pl.ds(e0, KD)], idx)
        pltpu.sync_copy(ones, degsp.at[idx], add=True)
        return 0
    lax.fori_loop(0, nch, _chunk, 0)
    plsc.subcore_barrier()

    for q in range(5):
        rows = pl.ds(sid * rows_pt + q * zch, zch)
        pltpu.sync_copy(degsp.at[rows], zb)
        pltpu.sync_copy(zb, out_ref.at[cid, rows])


def kernel(node_features, edge_index, edge_attr, emb,
           msg_W1, msg_b1, msg_W2, msg_b2,
           upd_W1, upd_b1, upd_W2, upd_b2,
           out_W1, out_b1, out_W2, out_b2):
    n = node_features.shape[0]
    e = edge_index.shape[1]
    nlayers, hdim, in_dim = msg_W1.shape
    ndim = emb.shape[1]
    nrbf = in_dim - 2 * ndim
    assert ndim == 128 and hdim == 256
    assert n % (BN * 2) == 0 and e % (EB * 2) == 0
    assert (e // NS) % K == 0 and (e // (NC * NS)) % KD == 0
    rows_pt = -(-n // NS)
    rows_pt += (-rows_pt) % 40
    npad = NS * rows_pt

    src = edge_index[0].astype(jnp.int32)
    dst = edge_index[1].astype(jnp.int32)
    nf = node_features.astype(jnp.int32).reshape(n, 1)
    vpad = ((emb.shape[0] + 7) // 8) * 8
    emb_p = jnp.pad(emb, ((0, vpad - emb.shape[0]), (0, 0)))

    f32 = jnp.float32
    grid_n = n // BN

    h, t = pl.pallas_call(
        _embed_prep_body,
        grid=(grid_n,),
        in_specs=[pl.BlockSpec((BN, 1), lambda i: (i, 0)),
                  pl.BlockSpec((vpad, ndim), lambda i: (0, 0)),
                  pl.BlockSpec((hdim, hdim), lambda i: (0, 0))],
        out_specs=[pl.BlockSpec((BN, ndim), lambda i: (i, 0)),
                   pl.BlockSpec((4, BN, ndim), lambda i: (0, i, 0))],
        out_shape=[jax.ShapeDtypeStruct((n, ndim), f32),
                   jax.ShapeDtypeStruct((4, n, ndim), f32)],
        name="embed_prep",
    )(nf, emb_p, msg_W1[0, :, :2 * ndim])

    w1c = msg_W1[:, :, 2 * ndim:]
    r_all = pl.pallas_call(
        functools.partial(_rmat_body, nrbf=nrbf),
        grid=(nlayers, e // EB),
        in_specs=[pl.BlockSpec((EB, 3), lambda i, j: (j, 0)),
                  pl.BlockSpec((1, hdim, nrbf), lambda i, j: (i, 0, 0)),
                  pl.BlockSpec((1, 1, hdim), lambda i, j: (i, 0, 0))],
        out_specs=pl.BlockSpec((1, 2, EB, 128), lambda i, j: (i, 0, j, 0)),
        out_shape=jax.ShapeDtypeStruct((nlayers, 2, e, 128), f32),
        name="rmat",
    )(edge_attr, w1c, msg_b1.reshape(nlayers, 1, hdim))

    deg_mesh = plsc.VectorSubcoreMesh(core_axis_name="c", subcore_axis_name="s",
                                      num_cores=NC, num_subcores=NS)
    degs = pl.kernel(
        functools.partial(_deg_body, n=n, e=e, npad=npad),
        out_type=jax.ShapeDtypeStruct((NC, npad, 128), f32),
        mesh=deg_mesh,
        scratch_types=[
            pltpu.VMEM((rows_pt // 5, 128), f32),
            pltpu.VMEM((KD, 128), f32),
            pltpu.VMEM((KD,), jnp.int32),
            pltpu.VMEM_SHARED((npad, 128), f32),
        ],
        name="deg",
    )(dst)

    idx2 = (edge_index.astype(jnp.int32)
            .reshape(2, e // K, K).transpose(1, 0, 2).reshape(2 * e))

    for i in range(nlayers):
        t_flat = t.reshape(4 * n, ndim)

        mesh = plsc.VectorSubcoreMesh(core_axis_name="c",
                                      subcore_axis_name="s",
                                      num_cores=NC, num_subcores=NS)
        agg = pl.kernel(
            functools.partial(_edge_body, n=n, e=e, npad=npad),
            out_type=jax.ShapeDtypeStruct((NC, npad, 128), f32),
            mesh=mesh,
            scratch_types=[
                [pltpu.VMEM((2 * K,), jnp.int32)] * 3,
                [pltpu.VMEM((K,), jnp.int32)] * 3,
                [pltpu.VMEM((2 * K, 128), f32)] * 3,
                [pltpu.VMEM((K, 128), f32)] * 3,
                pltpu.VMEM_SHARED((npad, 128), f32),
                [pltpu.SemaphoreType.DMA] * 3,
                [pltpu.SemaphoreType.DMA] * 3,
            ],
            name="edges",
        )(t_flat, r_all[i], idx2)

        last = i == nlayers - 1
        upd_in_specs = [
            pl.BlockSpec((BN, ndim), lambda j: (j, 0)),
            pl.BlockSpec((NC, BN, 128), lambda j: (0, j, 0)),
            pl.BlockSpec((NC, BN, 128), lambda j: (0, j, 0)),
            pl.BlockSpec((ndim, hdim), lambda j: (0, 0)),
            pl.BlockSpec((1, ndim), lambda j: (0, 0)),
            pl.BlockSpec((hdim, hdim), lambda j: (0, 0)),
            pl.BlockSpec((1, hdim), lambda j: (0, 0)),
            pl.BlockSpec((ndim, hdim), lambda j: (0, 0)),
            pl.BlockSpec((1, ndim), lambda j: (0, 0)),
        ]
        upd_args = [h, agg, degs, msg_W2[i], msg_b2[i].reshape(1, -1),
                    upd_W1[i], upd_b1[i].reshape(1, -1),
                    upd_W2[i], upd_b2[i].reshape(1, -1)]
        if last:
            h = pl.pallas_call(
                _update_body,
                grid=(grid_n,),
                in_specs=upd_in_specs,
                out_specs=pl.BlockSpec((BN, ndim), lambda j: (j, 0)),
                out_shape=jax.ShapeDtypeStruct((n, ndim), f32),
                name="update",
            )(*upd_args)
        else:
            h, t = pl.pallas_call(
                _update_body,
                grid=(grid_n,),
                in_specs=upd_in_specs + [
                    pl.BlockSpec((hdim, hdim), lambda j: (0, 0))],
                out_specs=[pl.BlockSpec((BN, ndim), lambda j: (j, 0)),
                           pl.BlockSpec((4, BN, ndim), lambda j: (0, j, 0))],
                out_shape=[jax.ShapeDtypeStruct((n, ndim), f32),
                           jax.ShapeDtypeStruct((4, n, ndim), f32)],
                name="update_prep",
            )(*upd_args, msg_W1[i + 1, :, :2 * ndim])

    half = out_W1.shape[0]
    energy = pl.pallas_call(
        functools.partial(_energy_body, n=n),
        grid=(grid_n,),
        in_specs=[pl.BlockSpec((BN, ndim), lambda j: (j, 0)),
                  pl.BlockSpec((half, ndim), lambda j: (0, 0)),
                  pl.BlockSpec((1, half), lambda j: (0, 0)),
                  pl.BlockSpec((1, half), lambda j: (0, 0)),
                  pl.BlockSpec((1, 1), lambda j: (0, 0))],
        out_specs=pl.BlockSpec((1, 1), lambda j: (0, 0)),
        out_shape=jax.ShapeDtypeStruct((1, 1), f32),
        name="energy",
    )(h, out_W1, out_b1.reshape(1, -1), out_W2, out_b2.reshape(1, 1))
    return energy[0, 0]


# speedup vs baseline: 1.1142x; 1.1142x over previous
import functools

import jax
import jax.numpy as jnp
from jax import lax
from jax.experimental import pallas as pl
from jax.experimental.pallas import tpu as pltpu
from jax.experimental.pallas import tpu_sc as plsc

CUTOFF = 5.0

NC = 2
NS = 16
BN = 1000
EB = 2000
K = 32
KD = 40


def _prep_mats(h, w, t_ref):
    dot = functools.partial(jnp.dot, preferred_element_type=jnp.float32)
    t_ref[0] = dot(h, w[:128, :128].T)
    t_ref[1] = dot(h, w[:128, 128:].T)
    t_ref[2] = dot(h, w[128:, :128].T)
    t_ref[3] = dot(h, w[128:, 128:].T)


def _embed_prep_body(nf_ref, emb_ref, w_ref, h_ref, t_ref):
    nf = nf_ref[...]
    v = emb_ref.shape[0]
    io = lax.broadcasted_iota(jnp.int32, (nf.shape[0], v), 1)
    onehot = jnp.where(nf == io, 1.0, 0.0).astype(jnp.float32)
    h = jnp.dot(onehot, emb_ref[...], preferred_element_type=jnp.float32)
    h_ref[...] = h
    _prep_mats(h, w_ref[...], t_ref)


def _rmat_body(ea_ref, w1c_ref, b1_ref, out_ref, *, nrbf):
    ea = ea_ref[...]
    d = jnp.sqrt(jnp.sum(ea * ea, axis=1, keepdims=True))
    mu = lax.broadcasted_iota(jnp.int32, (1, nrbf), 1).astype(jnp.float32) * (
        CUTOFF / (nrbf - 1))
    sigma = CUTOFF / nrbf
    rbf = jnp.exp(-((d - mu) ** 2) * (1.0 / (2.0 * sigma * sigma)))
    w1c = w1c_ref[0]
    r = jnp.dot(rbf, w1c.T, preferred_element_type=jnp.float32) + b1_ref[0]
    out_ref[0, 0] = r[:, :128]
    out_ref[0, 1] = r[:, 128:]


def _prep_body(h_ref, w_ref, out_ref):
    h = h_ref[...]
    w = w_ref[...]
    dot = functools.partial(jnp.dot, preferred_element_type=jnp.float32)
    out_ref[0] = dot(h, w[:128, :128].T)
    out_ref[1] = dot(h, w[:128, 128:].T)
    out_ref[2] = dot(h, w[128:, :128].T)
    out_ref[3] = dot(h, w[128:, 128:].T)


def _silu(x):
    return x * (1.0 / (1.0 + jnp.exp(-x)))


def _update_body(h_ref, agg_ref, degs_ref, w2_ref, b2_ref, uw1_ref, ub1_ref,
                 uw2_ref, ub2_ref, out_ref, *extra):
    h = h_ref[...]
    a0 = agg_ref[0]
    a1 = agg_ref[1]
    deg = degs_ref[0][:, 0:1] + degs_ref[1][:, 0:1]
    dot = functools.partial(jnp.dot, preferred_element_type=jnp.float32)
    w2 = w2_ref[...]
    agg = (dot(jnp.concatenate([a0, a1], axis=1), w2.T)
           + deg * b2_ref[...])
    uw1 = uw1_ref[...]
    pre = (dot(jnp.concatenate([h, agg], axis=1), uw1.T) + ub1_ref[...])
    hid = _silu(pre)
    hnew = h + dot(hid, uw2_ref[...].T) + ub2_ref[...]
    if extra:
        hout_ref, tout_ref = extra
        hout_ref[...] = hnew
        _prep_mats(hnew, out_ref[...], tout_ref)
    else:
        out_ref[...] = hnew


def _energy_body(h_ref, ow1_ref, ob1_ref, ow2_ref, ob2_ref, out_ref, *, n):
    dot = functools.partial(jnp.dot, preferred_element_type=jnp.float32)
    pre = dot(h_ref[...], ow1_ref[...].T) + ob1_ref[...]
    hid = _silu(pre)
    ae = dot(hid, ow2_ref[...].T)
    @pl.when(pl.program_id(0) == 0)
    def _():
        out_ref[...] = ob2_ref[...] * float(n)
    out_ref[...] += jnp.sum(ae).reshape(1, 1)


def _edge_body(t_ref, r_ref, idx2_ref, out_ref,
               ib, idst, gbuf, rbuf, agg, semg, semr, *, n, e, npad):
    cid = lax.axis_index("c")
    sid = lax.axis_index("s")
    ep = e // NS
    nch = ep // K
    rows_pt = npad // NS
    nzc = rows_pt // (2 * K)

    def _zrow(i, _):
        for v in range(8):
            gbuf[0][i, pl.ds(v * 16, 16)] = jnp.zeros((16,), jnp.float32)
        return 0
    lax.fori_loop(0, 2 * K, _zrow, 0)
    for q in range(nzc):
        pltpu.sync_copy(gbuf[0],
                        agg.at[pl.ds(sid * rows_pt + q * 2 * K, 2 * K)])
    plsc.subcore_barrier()

    off_a = cid * (2 * n)
    off_b = off_a + n
    base = sid * ep

    def _issue(j, b):
        g0 = pl.multiple_of((base // K + j) * 2 * K, 16)
        pltpu.sync_copy(idx2_ref.at[pl.ds(g0, 2 * K)], ib[b])
        for t in range(K // 16):
            sl = pl.ds(t * 16, 16)
            idst[b][sl] = ib[b][pl.ds(K + t * 16, 16)]
        for t in range(2 * K // 16):
            sl = pl.ds(t * 16, 16)
            ib[b][sl] = ib[b][sl] + (off_a if t < K // 16 else off_b)
        pltpu.async_copy(t_ref.at[ib[b]], gbuf[b], semg[b])
        e0 = pl.multiple_of(base + j * K, 16)
        pltpu.async_copy(r_ref.at[cid, pl.ds(e0, K)], rbuf[b], semr[b])

    def _process(b):
        pltpu.make_async_copy(t_ref.at[ib[b]], gbuf[b], semg[b]).wait()
        pltpu.make_async_copy(r_ref.at[cid, pl.ds(0, K)], rbuf[b],
                              semr[b]).wait()

        def _row(i, _):
            for v in range(8):
                sl = pl.ds(v * 16, 16)
                x = gbuf[b][i, sl] + gbuf[b][K + i, sl] + rbuf[b][i, sl]
                gbuf[b][i, sl] = x * (1.0 / (1.0 + jnp.exp(-x)))
            return 0
        lax.fori_loop(0, K, _row, 0)
        pltpu.sync_copy(gbuf[b].at[pl.ds(0, K)], agg.at[idst[b]], add=True)

    _issue(0, 0)
    _issue(1, 1)

    def _grp(jj, _):
        j = jj * 3
        for p in range(3):
            c = j + p

            @pl.when(c + 2 < nch)
            def _():
                _issue(c + 2, (p + 2) % 3)

            @pl.when(c < nch)
            def _():
                _process(p)
        return 0
    lax.fori_loop(0, (nch + 2) // 3, _grp, 0)
    plsc.subcore_barrier()

    for q in range(nzc):
        rows = pl.ds(sid * rows_pt + q * 2 * K, 2 * K)
        pltpu.sync_copy(agg.at[rows], gbuf[0])
        pltpu.sync_copy(gbuf[0], out_ref.at[cid, rows])


def _deg_body(dst_ref, out_ref, zb, ones, idx, degsp, *, n, e, npad):
    cid = lax.axis_index("c")
    sid = lax.axis_index("s")
    ept = e // (NC * NS)
    nch = ept // KD
    rows_pt = npad // NS
    zch = rows_pt // 5

    def _z(i, _):
        for v in range(8):
            zb[i, pl.ds(v * 16, 16)] = jnp.zeros((16,), jnp.float32)
        return 0
    lax.fori_loop(0, zch, _z, 0)

    def _o(i, _):
        for v in range(8):
            ones[i, pl.ds(v * 16, 16)] = jnp.full((16,), 1.0, jnp.float32)
        return 0
    lax.fori_loop(0, KD, _o, 0)

    for q in range(5):
        pltpu.sync_copy(zb, degsp.at[pl.ds(sid * rows_pt + q * zch, zch)])
    plsc.subcore_barrier()

    base = cid * (e // NC) + sid * ept

    def _chunk(j, _):
        e0 = pl.multiple_of(base + j * KD, 8)
        pltpu.sync_copy(dst_ref.at[pl.ds(e0, KD)], idx)
        pltpu.sync_copy(ones, degsp.at[idx], add=True)
        return 0
    lax.fori_loop(0, nch, _chunk, 0)
    plsc.subcore_barrier()

    for q in range(5):
        rows = pl.ds(sid * rows_pt + q * zch, zch)
        pltpu.sync_copy(degsp.at[rows], zb)
        pltpu.sync_copy(zb, out_ref.at[cid, rows])


def kernel(node_features, edge_index, edge_attr, emb,
           msg_W1, msg_b1, msg_W2, msg_b2,
           upd_W1, upd_b1, upd_W2, upd_b2,
           out_W1, out_b1, out_W2, out_b2):
    n = node_features.shape[0]
    e = edge_index.shape[1]
    nlayers, hdim, in_dim = msg_W1.shape
    ndim = emb.shape[1]
    nrbf = in_dim - 2 * ndim
    assert ndim == 128 and hdim == 256
    assert n % (BN * 2) == 0 and e % (EB * 2) == 0
    assert (e // NS) % K == 0 and (e // (NC * NS)) % KD == 0
    rows_pt = -(-n // NS)
    rows_pt += (-rows_pt) % 40
    npad = NS * rows_pt

    src = edge_index[0].astype(jnp.int32)
    dst = edge_index[1].astype(jnp.int32)
    nf = node_features.astype(jnp.int32).reshape(n, 1)
    vpad = ((emb.shape[0] + 7) // 8) * 8
    emb_p = jnp.pad(emb, ((0, vpad - emb.shape[0]), (0, 0)))

    f32 = jnp.float32
    grid_n = n // BN

    h, t = pl.pallas_call(
        _embed_prep_body,
        grid=(grid_n,),
        in_specs=[pl.BlockSpec((BN, 1), lambda i: (i, 0)),
                  pl.BlockSpec((vpad, ndim), lambda i: (0, 0)),
                  pl.BlockSpec((hdim, hdim), lambda i: (0, 0))],
        out_specs=[pl.BlockSpec((BN, ndim), lambda i: (i, 0)),
                   pl.BlockSpec((4, BN, ndim), lambda i: (0, i, 0))],
        out_shape=[jax.ShapeDtypeStruct((n, ndim), f32),
                   jax.ShapeDtypeStruct((4, n, ndim), f32)],
        name="embed_prep",
    )(nf, emb_p, msg_W1[0, :, :2 * ndim])

    w1c = msg_W1[:, :, 2 * ndim:]
    r_all = pl.pallas_call(
        functools.partial(_rmat_body, nrbf=nrbf),
        grid=(nlayers, e // EB),
        in_specs=[pl.BlockSpec((EB, 3), lambda i, j: (j, 0)),
                  pl.BlockSpec((1, hdim, nrbf), lambda i, j: (i, 0, 0)),
                  pl.BlockSpec((1, 1, hdim), lambda i, j: (i, 0, 0))],
        out_specs=pl.BlockSpec((1, 2, EB, 128), lambda i, j: (i, 0, j, 0)),
        out_shape=jax.ShapeDtypeStruct((nlayers, 2, e, 128), f32),
        name="rmat",
    )(edge_attr, w1c, msg_b1.reshape(nlayers, 1, hdim))

    deg_mesh = plsc.VectorSubcoreMesh(core_axis_name="c", subcore_axis_name="s",
                                      num_cores=NC, num_subcores=NS)
    degs = pl.kernel(
        functools.partial(_deg_body, n=n, e=e, npad=npad),
        out_type=jax.ShapeDtypeStruct((NC, npad, 128), f32),
        mesh=deg_mesh,
        scratch_types=[
            pltpu.VMEM((rows_pt // 5, 128), f32),
            pltpu.VMEM((KD, 128), f32),
            pltpu.VMEM((KD,), jnp.int32),
            pltpu.VMEM_SHARED((npad, 128), f32),
        ],
        name="deg",
    )(dst)

    idx2 = (edge_index.astype(jnp.int32)
            .reshape(2, e // K, K).transpose(1, 0, 2).reshape(2 * e))

    for i in range(nlayers):
        t_flat = t.reshape(4 * n, ndim)

        mesh = plsc.VectorSubcoreMesh(core_axis_name="c",
                                      subcore_axis_name="s",
                                      num_cores=NC, num_subcores=NS)
        agg = pl.kernel(
            functools.partial(_edge_body, n=n, e=e, npad=npad),
            out_type=jax.ShapeDtypeStruct((NC, npad, 128), f32),
            mesh=mesh,
            scratch_types=[
                [pltpu.VMEM((2 * K,), jnp.int32)] * 3,
                [pltpu.VMEM((K,), jnp.int32)] * 3,
                [pltpu.VMEM((2 * K, 128), f32)] * 3,
                [pltpu.VMEM((K, 128), f32)] * 3,
                pltpu.VMEM_SHARED((npad, 128), f32),
                [pltpu.SemaphoreType.DMA] * 3,
                [pltpu.SemaphoreType.DMA] * 3,
            ],
            name="edges",
        )(t_flat, r_all[i], idx2)

        last = i == nlayers - 1
        upd_in_specs = [
            pl.BlockSpec((BN, ndim), lambda j: (j, 0)),
            pl.BlockSpec((NC, BN, 128), lambda j: (0, j, 0)),
            pl.BlockSpec((NC, BN, 128), lambda j: (0, j, 0)),
            pl.BlockSpec((ndim, hdim), lambda j: (0, 0)),
            pl.BlockSpec((1, ndim), lambda j: (0, 0)),
            pl.BlockSpec((hdim, hdim), lambda j: (0, 0)),
            pl.BlockSpec((1, hdim), lambda j: (0, 0)),
            pl.BlockSpec((ndim, hdim), lambda j: (0, 0)),
            pl.BlockSpec((1, ndim), lambda j: (0, 0)),
        ]
        upd_args = [h, agg, degs, msg_W2[i], msg_b2[i].reshape(1, -1),
                    upd_W1[i], upd_b1[i].reshape(1, -1),
                    upd_W2[i], upd_b2[i].reshape(1, -1)]
        if last:
            h = pl.pallas_call(
                _update_body,
                grid=(grid_n,),
                in_specs=upd_in_specs,
                out_specs=pl.BlockSpec((BN, ndim), lambda j: (j, 0)),
                out_shape=jax.ShapeDtypeStruct((n, ndim), f32),
                name="update",
            )(*upd_args)
        else:
            h, t = pl.pallas_call(
                _update_body,
                grid=(grid_n,),
                in_specs=upd_in_specs + [
                    pl.BlockSpec((hdim, hdim), lambda j: (0, 0))],
                out_specs=[pl.BlockSpec((BN, ndim), lambda j: (j, 0)),
                           pl.BlockSpec((4, BN, ndim), lambda j: (0, j, 0))],
                out_shape=[jax.ShapeDtypeStruct((n, ndim), f32),
                           jax.ShapeDtypeStruct((4, n, ndim), f32)],
                name="update_prep",
            )(*upd_args, msg_W1[i + 1, :, :2 * ndim])

    half = out_W1.shape[0]
    energy = pl.pallas_call(
        functools.partial(_energy_body, n=n),
        grid=(grid_n,),
        in_specs=[pl.BlockSpec((BN, ndim), lambda j: (j, 0)),
                  pl.BlockSpec((half, ndim), lambda j: (0, 0)),
                  pl.BlockSpec((1, half), lambda j: (0, 0)),
                  pl.BlockSpec((1, half), lambda j: (0, 0)),
                  pl.BlockSpec((1, 1), lambda j: (0, 0))],
        out_specs=pl.BlockSpec((1, 1), lambda j: (0, 0)),
        out_shape=jax.ShapeDtypeStruct((1, 1), f32),
        name="energy",
    )(h, out_W1, out_b1.reshape(1, -1), out_W2, out_b2.reshape(1, 1))
    return energy[0, 0]
